# baseline (speedup 1.0000x reference)
import functools

import jax
import jax.numpy as jnp
from jax import lax
from jax.experimental import pallas as pl
from jax.experimental.pallas import tpu as pltpu
from jax.experimental.pallas import tpu_sc as plsc

F = 26
V = 100000
D = 32
B = 16384
N = B * F

NC = 2
NS = 16
L = 16
NW = NC * NS
PER_W = N // NW
CHUNK = 1024
GSZ = 128

_mesh = plsc.VectorSubcoreMesh(core_axis_name="c", subcore_axis_name="s")


@functools.partial(
    pl.kernel,
    mesh=_mesh,
    out_type=jax.ShapeDtypeStruct((N, D), jnp.float32),
    scratch_types=[
        pltpu.VMEM((CHUNK,), jnp.int32),
        pltpu.VMEM((CHUNK, D), jnp.float32),
        pltpu.SemaphoreType.DMA,
    ],
    compiler_params=pltpu.CompilerParams(use_tc_tiling_on_sc=False),
)
def _gather_rows(idx_hbm, table_hbm, out_hbm, idx_v, rows_v, gsem):
    wid = lax.axis_index("s") * NC + lax.axis_index("c")
    base = wid * PER_W

    @pl.loop(0, PER_W, step=CHUNK)
    def _(c):
        start = base + c
        pltpu.sync_copy(idx_hbm.at[pl.ds(start, CHUNK)], idx_v)

        @pl.loop(0, CHUNK, step=L)
        def _(j):
            lanes = (start + j) + lax.iota(jnp.int32, L)
            idx_v[pl.ds(j, L)] = idx_v[pl.ds(j, L)] + (lanes % F) * V

        copies = [
            pltpu.async_copy(
                table_hbm.at[idx_v.at[pl.ds(g * GSZ, GSZ)]],
                rows_v.at[pl.ds(g * GSZ, GSZ)],
                gsem,
            )
            for g in range(CHUNK // GSZ)
        ]
        for cp in copies:
            cp.wait()

        pltpu.sync_copy(rows_v, out_hbm.at[pl.ds(start, CHUNK)])


def kernel(X, tables):
    idx_flat = X.reshape(N)
    table_flat = tables.reshape(F * V, D)
    out = _gather_rows(idx_flat, table_flat)
    return out.reshape(B, F * D)


# speedup vs baseline: 1.0108x; 1.0108x over previous
import functools

import jax
import jax.numpy as jnp
from jax import lax
from jax.experimental import pallas as pl
from jax.experimental.pallas import tpu as pltpu
from jax.experimental.pallas import tpu_sc as plsc

F = 26
V = 100000
D = 32
B = 16384
N = B * F

NC = 2
NS = 16
L = 16
NW = NC * NS
PER_W = N // NW
CHUNK = 1664
GSZ = 128
PAIR = 2 * CHUNK

_mesh = plsc.VectorSubcoreMesh(core_axis_name="c", subcore_axis_name="s")


@functools.partial(
    pl.kernel,
    mesh=_mesh,
    out_type=jax.ShapeDtypeStruct((N, D), jnp.float32),
    scratch_types=[
        pltpu.VMEM((CHUNK,), jnp.int32),
        pltpu.VMEM((CHUNK,), jnp.int32),
        pltpu.VMEM((CHUNK,), jnp.int32),
        pltpu.VMEM((CHUNK, D), jnp.float32),
        pltpu.VMEM((CHUNK, D), jnp.float32),
        pltpu.SemaphoreType.DMA,
        pltpu.SemaphoreType.DMA,
        pltpu.SemaphoreType.DMA,
        pltpu.SemaphoreType.DMA,
        pltpu.SemaphoreType.DMA,
        pltpu.SemaphoreType.DMA,
    ],
    compiler_params=pltpu.CompilerParams(use_tc_tiling_on_sc=False),
)
def _gather_rows(idx_hbm, table_hbm, out_hbm, idx_a, idx_b, off_v,
                 rows_a, rows_b, isem_a, isem_b, gsem_a, gsem_b,
                 osem_a, osem_b):
    wid = lax.axis_index("s") * NC + lax.axis_index("c")
    base = wid * PER_W

    @pl.loop(0, CHUNK, step=L)
    def _(j):
        off_v[pl.ds(j, L)] = ((j + lax.iota(jnp.int32, L)) % F) * V

    pltpu.async_copy(idx_hbm.at[pl.ds(base, CHUNK)], idx_a, isem_a)
    pltpu.async_copy(idx_hbm.at[pl.ds(base + CHUNK, CHUNK)], idx_b, isem_b)

    def fire_gathers(c, idx_v, rows_v, isem, osem, gsem, first):
        start = base + c
        pltpu.make_async_copy(idx_hbm.at[pl.ds(start, CHUNK)], idx_v,
                              isem).wait()

        @pl.loop(0, CHUNK, step=L)
        def _(j):
            idx_v[pl.ds(j, L)] = idx_v[pl.ds(j, L)] + off_v[pl.ds(j, L)]

        @pl.when(jnp.logical_not(first))
        def _():
            pltpu.make_async_copy(rows_v, out_hbm.at[pl.ds(base, CHUNK)],
                                  osem).wait()

        return [
            pltpu.async_copy(
                table_hbm.at[idx_v.at[pl.ds(g * GSZ, GSZ)]],
                rows_v.at[pl.ds(g * GSZ, GSZ)],
                gsem,
            )
            for g in range(CHUNK // GSZ)
        ]

    def drain_and_writeback(c, idx_v, rows_v, isem, osem, gathers, nxt):
        for cp in gathers:
            cp.wait()
        pltpu.async_copy(rows_v, out_hbm.at[pl.ds(base + c, CHUNK)], osem)

        @pl.when(nxt + CHUNK <= PER_W)
        def _():
            pltpu.async_copy(idx_hbm.at[pl.ds(base + nxt, CHUNK)], idx_v,
                             isem)

    @pl.loop(0, PER_W, step=PAIR)
    def _(c):
        first = c == 0
        ga = fire_gathers(c, idx_a, rows_a, isem_a, osem_a, gsem_a, first)
        gb = fire_gathers(c + CHUNK, idx_b, rows_b, isem_b, osem_b, gsem_b,
                          first)
        drain_and_writeback(c, idx_a, rows_a, isem_a, osem_a, ga, c + PAIR)
        drain_and_writeback(c + CHUNK, idx_b, rows_b, isem_b, osem_b, gb,
                            c + PAIR + CHUNK)

    pltpu.make_async_copy(rows_a, out_hbm.at[pl.ds(base, CHUNK)],
                          osem_a).wait()
    pltpu.make_async_copy(rows_b, out_hbm.at[pl.ds(base, CHUNK)],
                          osem_b).wait()


def kernel(X, tables):
    idx_flat = X.reshape(N)
    table_flat = tables.reshape(F * V, D)
    out = _gather_rows(idx_flat, table_flat)
    return out.reshape(B, F * D)


# speedup vs baseline: 2.9347x; 2.9033x over previous
import dataclasses
import functools

import jax
import jax.numpy as jnp
from jax import lax
from jax.experimental import pallas as pl
from jax.experimental.pallas import tpu as pltpu
from jax.experimental.pallas import tpu_sc as plsc

F = 26
V = 100000
D = 32
B = 16384
L = 16
NC = 2
Q = 4096
NQ = B // Q

_mesh = plsc.VectorSubcoreMesh(core_axis_name="c", subcore_axis_name="s")

_cp = pltpu.CompilerParams(use_tc_tiling_on_sc=True)
if "needs_layout_passes" in pltpu.CompilerParams.__dataclass_fields__:
    _cp = dataclasses.replace(_cp, needs_layout_passes=False)


@functools.partial(
    pl.kernel,
    mesh=_mesh,
    out_type=jax.ShapeDtypeStruct((F * D, B), jnp.float32),
    scratch_types=[
        pltpu.VMEM((1, B), jnp.int32),
        pltpu.VMEM((1, V), jnp.float32),
        pltpu.VMEM((1, Q), jnp.float32),
        pltpu.VMEM((1, Q), jnp.float32),
        pltpu.SemaphoreType.DMA,
        pltpu.SemaphoreType.DMA,
    ],
    compiler_params=_cp,
)
def _field_gather(xt_hbm, t2_hbm, out_hbm, idx_v, vec_v, out_a, out_b,
                  wsem_a, wsem_b):
    wid = lax.axis_index("s") * NC + lax.axis_index("c")

    @pl.when(wid < F)
    def _():
        f = wid
        pltpu.sync_copy(xt_hbm.at[pl.ds(f, 1), :], idx_v)

        @pl.loop(0, D)
        def _(d):
            pltpu.sync_copy(t2_hbm.at[pl.ds(f * D + d, 1), :], vec_v)

            for q in range(NQ):
                out_v, wsem = (out_a, wsem_a) if q % 2 == 0 else (out_b,
                                                                  wsem_b)
                if q < 2:
                    @pl.when(d > 0)
                    def _():
                        pltpu.make_async_copy(
                            out_v, out_hbm.at[pl.ds(0, 1), pl.ds(0, Q)],
                            wsem).wait()
                else:
                    pltpu.make_async_copy(
                        out_v, out_hbm.at[pl.ds(0, 1), pl.ds(0, Q)],
                        wsem).wait()

                @pl.loop(0, Q, step=L, unroll=8)
                def _(j):
                    iv = idx_v[0, pl.ds(q * Q + j, L)]
                    out_v[0, pl.ds(j, L)] = plsc.load_gather(
                        vec_v, [jnp.zeros((L,), jnp.int32), iv])

                pltpu.async_copy(
                    out_v,
                    out_hbm.at[pl.ds(f * D + d, 1), pl.ds(q * Q, Q)],
                    wsem)

        pltpu.make_async_copy(out_a, out_hbm.at[pl.ds(0, 1), pl.ds(0, Q)],
                              wsem_a).wait()
        pltpu.make_async_copy(out_b, out_hbm.at[pl.ds(0, 1), pl.ds(0, Q)],
                              wsem_b).wait()


def kernel(X, tables):
    xt = X.T
    tt = jnp.transpose(tables, (0, 2, 1))
    t2 = tt.reshape(F * D, V)
    out_t = _field_gather(xt, t2)
    return out_t.T


# speedup vs baseline: 3.4372x; 1.1712x over previous
import dataclasses
import functools

import jax
import jax.numpy as jnp
from jax import lax
from jax.experimental import pallas as pl
from jax.experimental.pallas import tpu as pltpu
from jax.experimental.pallas import tpu_sc as plsc

F = 26
V = 100000
D = 32
B = 16384
L = 16
NC = 2
NW = 32
PER_W = F * D // NW
Q = 4096
NQ = B // Q
VH = 50048

_mesh = plsc.VectorSubcoreMesh(core_axis_name="c", subcore_axis_name="s")

_cp = pltpu.CompilerParams(use_tc_tiling_on_sc=True)
if "needs_layout_passes" in pltpu.CompilerParams.__dataclass_fields__:
    _cp = dataclasses.replace(_cp, needs_layout_passes=False)


@functools.partial(
    pl.kernel,
    mesh=_mesh,
    out_type=jax.ShapeDtypeStruct((F * D, B), jnp.float32),
    scratch_types=[
        pltpu.VMEM((1, B), jnp.int32),
        pltpu.VMEM((1, V), jnp.float32),
        pltpu.VMEM((1, Q), jnp.float32),
        pltpu.VMEM((1, Q), jnp.float32),
        pltpu.SemaphoreType.DMA,
        pltpu.SemaphoreType.DMA,
        pltpu.SemaphoreType.DMA,
        pltpu.SemaphoreType.DMA,
    ],
    compiler_params=_cp,
)
def _field_gather(xt_hbm, t2_hbm, out_hbm, idx_v, vec_v, out_a, out_b,
                  vsem_a, vsem_b, wsem_a, wsem_b):
    wid = lax.axis_index("s") * NC + lax.axis_index("c")
    fd0 = wid * PER_W

    @pl.loop(0, PER_W)
    def _(k):
        fd = fd0 + k
        f = lax.shift_right_logical(fd, 5)
        pltpu.async_copy(t2_hbm.at[pl.ds(fd, 1), pl.ds(0, VH)],
                         vec_v.at[:, pl.ds(0, VH)], vsem_a)
        pltpu.async_copy(t2_hbm.at[pl.ds(fd, 1), pl.ds(VH, V - VH)],
                         vec_v.at[:, pl.ds(VH, V - VH)], vsem_b)

        @pl.when((k == 0) | (lax.bitwise_and(fd, D - 1) == 0))
        def _():
            pltpu.sync_copy(xt_hbm.at[pl.ds(f, 1), :], idx_v)

        pltpu.make_async_copy(t2_hbm.at[pl.ds(0, 1), pl.ds(0, VH)],
                              vec_v.at[:, pl.ds(0, VH)], vsem_a).wait()
        pltpu.make_async_copy(t2_hbm.at[pl.ds(0, 1), pl.ds(VH, V - VH)],
                              vec_v.at[:, pl.ds(VH, V - VH)], vsem_b).wait()

        for q in range(NQ):
            out_v, wsem = (out_a, wsem_a) if q % 2 == 0 else (out_b, wsem_b)
            if q < 2:
                @pl.when(k > 0)
                def _():
                    pltpu.make_async_copy(
                        out_v, out_hbm.at[pl.ds(0, 1), pl.ds(0, Q)],
                        wsem).wait()
            else:
                pltpu.make_async_copy(
                    out_v, out_hbm.at[pl.ds(0, 1), pl.ds(0, Q)],
                    wsem).wait()

            @pl.loop(0, Q, step=L, unroll=8)
            def _(j):
                iv = idx_v[0, pl.ds(q * Q + j, L)]
                out_v[0, pl.ds(j, L)] = plsc.load_gather(vec_v.at[0], [iv])

            pltpu.async_copy(
                out_v, out_hbm.at[pl.ds(fd, 1), pl.ds(q * Q, Q)], wsem)

    pltpu.make_async_copy(out_a, out_hbm.at[pl.ds(0, 1), pl.ds(0, Q)],
                          wsem_a).wait()
    pltpu.make_async_copy(out_b, out_hbm.at[pl.ds(0, 1), pl.ds(0, Q)],
                          wsem_b).wait()


def kernel(X, tables):
    xt = X.T
    tt = jnp.transpose(tables, (0, 2, 1))
    t2 = tt.reshape(F * D, V)
    out_t = _field_gather(xt, t2)
    return out_t.T


# speedup vs baseline: 6.9838x; 2.0319x over previous
import dataclasses
import functools

import jax
import jax.numpy as jnp
from jax import lax
from jax.experimental import pallas as pl
from jax.experimental.pallas import tpu as pltpu
from jax.experimental.pallas import tpu_sc as plsc

F = 26
V = 100000
D = 32
B = 16384
L = 16
NC = 2
NW = 32
PER_W = F * D // NW
Q = 4096
NQ = B // Q
VH = 50048

_mesh = plsc.VectorSubcoreMesh(core_axis_name="c", subcore_axis_name="s")

_cp = pltpu.CompilerParams(use_tc_tiling_on_sc=True)
if "needs_layout_passes" in pltpu.CompilerParams.__dataclass_fields__:
    _cp = dataclasses.replace(_cp, needs_layout_passes=False)


@functools.partial(
    pl.kernel,
    mesh=_mesh,
    out_type=jax.ShapeDtypeStruct((F * D, B), jnp.float32),
    scratch_types=[
        pltpu.VMEM((1, B), jnp.int32),
        pltpu.VMEM((1, V), jnp.float32),
        pltpu.VMEM((1, Q), jnp.float32),
        pltpu.VMEM((1, Q), jnp.float32),
        pltpu.SemaphoreType.DMA,
        pltpu.SemaphoreType.DMA,
        pltpu.SemaphoreType.DMA,
        pltpu.SemaphoreType.DMA,
    ],
    compiler_params=_cp,
)
def _field_gather(xt_hbm, t2_hbm, out_hbm, idx_v, vec_v, out_a, out_b,
                  vsem_a, vsem_b, wsem_a, wsem_b):
    wid = lax.axis_index("s") * NC + lax.axis_index("c")
    fd0 = wid * PER_W

    @pl.loop(0, PER_W)
    def _(k):
        fd = fd0 + k
        f = lax.shift_right_logical(fd, 5)
        pltpu.async_copy(t2_hbm.at[pl.ds(fd, 1), pl.ds(0, VH)],
                         vec_v.at[:, pl.ds(0, VH)], vsem_a)
        pltpu.async_copy(t2_hbm.at[pl.ds(fd, 1), pl.ds(VH, V - VH)],
                         vec_v.at[:, pl.ds(VH, V - VH)], vsem_b)

        @pl.when((k == 0) | (lax.bitwise_and(fd, D - 1) == 0))
        def _():
            pltpu.sync_copy(xt_hbm.at[pl.ds(f, 1), :], idx_v)

        pltpu.make_async_copy(t2_hbm.at[pl.ds(0, 1), pl.ds(0, VH)],
                              vec_v.at[:, pl.ds(0, VH)], vsem_a).wait()
        pltpu.make_async_copy(t2_hbm.at[pl.ds(0, 1), pl.ds(VH, V - VH)],
                              vec_v.at[:, pl.ds(VH, V - VH)], vsem_b).wait()

        for q in range(NQ):
            out_v, wsem = (out_a, wsem_a) if q % 2 == 0 else (out_b, wsem_b)
            if q < 2:
                @pl.when(k > 0)
                def _():
                    pltpu.make_async_copy(
                        out_v, out_hbm.at[pl.ds(0, 1), pl.ds(0, Q)],
                        wsem).wait()
            else:
                pltpu.make_async_copy(
                    out_v, out_hbm.at[pl.ds(0, 1), pl.ds(0, Q)],
                    wsem).wait()

            @plsc.parallel_loop(0, Q, step=L, unroll=8)
            def _(j):
                iv = idx_v[0, pl.ds(q * Q + j, L)]
                out_v[0, pl.ds(j, L)] = plsc.load_gather(vec_v.at[0], [iv])

            pltpu.async_copy(
                out_v, out_hbm.at[pl.ds(fd, 1), pl.ds(q * Q, Q)], wsem)

    pltpu.make_async_copy(out_a, out_hbm.at[pl.ds(0, 1), pl.ds(0, Q)],
                          wsem_a).wait()
    pltpu.make_async_copy(out_b, out_hbm.at[pl.ds(0, 1), pl.ds(0, Q)],
                          wsem_b).wait()


def kernel(X, tables):
    xt = X.T
    tt = jnp.transpose(tables, (0, 2, 1))
    t2 = tt.reshape(F * D, V)
    out_t = _field_gather(xt, t2)
    return out_t.T
